# baseline (speedup 1.0000x reference)
import jax
import jax.numpy as jnp
from jax import lax
from jax.experimental import pallas as pl
from jax.experimental.pallas import tpu as pltpu
from jax.experimental.pallas import tpu_sc as plsc

D_DIM = 1024
CHUNK_ROWS = 16
CHUNK_W = CHUNK_ROWS * D_DIM
N_WORKERS = 32
UNROLL = 8
NBUF = 4


def _sc_body(x_hbm, e_hbm, o_hbm, *refs):
    ev = list(refs[0:2])
    xv = list(refs[2:6])
    esem = list(refs[6:8])
    xisem = list(refs[8:12])
    xosem = list(refs[12:16])

    wid = lax.axis_index("s") * 2 + lax.axis_index("c")
    rows = 4096 // N_WORKERS
    nch = rows // CHUNK_ROWS
    nsteps = nch * 4
    base_s = wid * rows

    def e_desc(k):
        s0 = base_s + k * CHUNK_ROWS
        return pltpu.make_async_copy(
            e_hbm.at[pl.ds(s0 * D_DIM, CHUNK_W)], ev[k % 2], esem[k % 2])

    def x_off(t):
        k, b = divmod(t, 4)
        s0 = base_s + k * CHUNK_ROWS
        return (b * 4096 + s0) * D_DIM

    def xin_desc(t):
        return pltpu.make_async_copy(
            x_hbm.at[pl.ds(x_off(t), CHUNK_W)], xv[t % NBUF], xisem[t % NBUF])

    def xout_desc(t):
        return pltpu.make_async_copy(
            xv[t % NBUF], o_hbm.at[pl.ds(x_off(t), CHUNK_W)], xosem[t % NBUF])

    e_desc(0).start()
    e_desc(1).start()
    xin_desc(0).start()
    xin_desc(1).start()

    for t in range(nsteps):
        k, b = divmod(t, 4)
        p = t % NBUF
        if b == 0:
            e_desc(k).wait()
        xin_desc(t).wait()
        xb, eb = xv[p], ev[k % 2]

        @plsc.parallel_loop(0, CHUNK_W, step=16, unroll=UNROLL)
        def add_body(i, xb=xb, eb=eb):
            sl = pl.ds(i, 16)
            xb[sl] = xb[sl] + eb[sl]
        xout_desc(t).start()
        if b == 3 and k + 2 < nch:
            e_desc(k + 2).start()
        if t + 2 < nsteps:
            if t - 2 >= 0:
                xout_desc(t - 2).wait()
            xin_desc(t + 2).start()

    xout_desc(nsteps - 2).wait()
    xout_desc(nsteps - 1).wait()


def kernel(x, embed_weight):
    B, S, D = x.shape
    xf = x.reshape(-1)
    ef = embed_weight.reshape(-1)
    mesh = plsc.VectorSubcoreMesh(core_axis_name="c", subcore_axis_name="s")
    scratch = (
        [pltpu.VMEM((CHUNK_W,), jnp.float32)] * 2
        + [pltpu.VMEM((CHUNK_W,), jnp.float32)] * 4
        + [pltpu.SemaphoreType.DMA] * 10
    )
    f = pl.kernel(
        _sc_body,
        out_type=jax.ShapeDtypeStruct((B * S * D,), jnp.float32),
        mesh=mesh,
        scratch_types=scratch,
    )
    return f(xf, ef).reshape(B, S, D)


# speedup vs baseline: 4.8478x; 4.8478x over previous
import jax
import jax.numpy as jnp
from jax.experimental import pallas as pl
from jax.experimental.pallas import tpu as pltpu

BLOCK_S = 2048


def _add_kernel(x_ref, e_ref, o_ref):
    o_ref[...] = x_ref[...] + e_ref[...][None, :, :]


def kernel(x, embed_weight):
    B, S, D = x.shape
    grid = (S // BLOCK_S, B)
    return pl.pallas_call(
        _add_kernel,
        grid=grid,
        in_specs=[
            pl.BlockSpec((1, BLOCK_S, D), lambda s, b: (b, s, 0)),
            pl.BlockSpec((BLOCK_S, D), lambda s, b: (s, 0)),
        ],
        out_specs=pl.BlockSpec((1, BLOCK_S, D), lambda s, b: (b, s, 0)),
        out_shape=jax.ShapeDtypeStruct((B, S, D), x.dtype),
        compiler_params=pltpu.CompilerParams(
            dimension_semantics=("parallel", "parallel")
        ),
    )(x, embed_weight)
